# speedup vs baseline: 2.9657x; 2.9657x over previous
import functools

import jax
import jax.numpy as jnp
from jax import lax
from jax.experimental import pallas as pl
from jax.experimental.pallas import tpu as pltpu
from jax.experimental.pallas import tpu_sc as plsc

NUM_CORES = 2
NUM_SUBCORES = 16
NUM_WORKERS = NUM_CORES * NUM_SUBCORES

CHUNK = 128


def _body(idx_hbm, table_hbm, out_hbm, idx_v, rows_v, gsem, b_per_w):
    wid = lax.axis_index("s") * NUM_CORES + lax.axis_index("c")
    base = wid * b_per_w
    pltpu.sync_copy(idx_hbm.at[pl.ds(base, b_per_w)], idx_v)

    n_chunks = b_per_w // CHUNK

    def step(j, carry):
        off = j * CHUNK
        idx_slice = idx_v.at[pl.ds(off, CHUNK)]
        pltpu.async_copy(table_hbm.at[idx_slice], rows_v, gsem).wait()
        pltpu.sync_copy(rows_v, out_hbm.at[pl.ds(base + off, CHUNK)])
        return carry

    lax.fori_loop(0, n_chunks, step, 0)


def kernel(x, weight):
    lead_shape = x.shape
    d = weight.shape[1]
    flat = x.reshape(-1).astype(jnp.int32)
    b = flat.shape[0]
    assert b % (NUM_WORKERS * CHUNK) == 0
    b_per_w = b // NUM_WORKERS

    mesh = plsc.VectorSubcoreMesh(core_axis_name="c", subcore_axis_name="s")
    k = pl.kernel(
        functools.partial(_body, b_per_w=b_per_w),
        out_type=jax.ShapeDtypeStruct((b, d), jnp.float32),
        mesh=mesh,
        scratch_types=[
            pltpu.VMEM((b_per_w,), jnp.int32),
            pltpu.VMEM((CHUNK, d), jnp.float32),
            pltpu.SemaphoreType.DMA,
        ],
    )
    out = k(flat, weight)
    return out.reshape(lead_shape + (d,))


# speedup vs baseline: 3.2897x; 1.1092x over previous
import functools

import jax
import jax.numpy as jnp
from jax import lax
from jax.experimental import pallas as pl
from jax.experimental.pallas import tpu as pltpu
from jax.experimental.pallas import tpu_sc as plsc

NUM_CORES = 2
NUM_SUBCORES = 16
NUM_WORKERS = NUM_CORES * NUM_SUBCORES

CHUNK = 640


def _body(idx_hbm, table_hbm, out_hbm, idx_v, rows_v, gsem, b_per_w):
    wid = lax.axis_index("s") * NUM_CORES + lax.axis_index("c")
    base = wid * b_per_w
    pltpu.sync_copy(idx_hbm.at[pl.ds(base, b_per_w)], idx_v)

    n_chunks = b_per_w // CHUNK

    def step(j, carry):
        off = j * CHUNK
        idx_slice = idx_v.at[pl.ds(off, CHUNK)]
        pltpu.async_copy(table_hbm.at[idx_slice], rows_v, gsem).wait()
        pltpu.sync_copy(rows_v, out_hbm.at[pl.ds(base + off, CHUNK)])
        return carry

    lax.fori_loop(0, n_chunks, step, 0)


def kernel(x, weight):
    lead_shape = x.shape
    d = weight.shape[1]
    flat = x.reshape(-1).astype(jnp.int32)
    b = flat.shape[0]
    assert b % NUM_WORKERS == 0
    b_per_w = b // NUM_WORKERS
    assert b_per_w % CHUNK == 0

    mesh = plsc.VectorSubcoreMesh(core_axis_name="c", subcore_axis_name="s")
    k = pl.kernel(
        functools.partial(_body, b_per_w=b_per_w),
        out_type=jax.ShapeDtypeStruct((b, d), jnp.float32),
        mesh=mesh,
        scratch_types=[
            pltpu.VMEM((b_per_w,), jnp.int32),
            pltpu.VMEM((CHUNK, d), jnp.float32),
            pltpu.SemaphoreType.DMA,
        ],
    )
    out = k(flat, weight)
    return out.reshape(lead_shape + (d,))


# speedup vs baseline: 3.3340x; 1.0135x over previous
import functools

import jax
import jax.numpy as jnp
from jax import lax
from jax.experimental import pallas as pl
from jax.experimental.pallas import tpu as pltpu
from jax.experimental.pallas import tpu_sc as plsc

NUM_CORES = 2
NUM_SUBCORES = 16
NUM_WORKERS = NUM_CORES * NUM_SUBCORES

CHUNK = 200
NBUF = 4
LOOK = 2


def _body(idx_hbm, table_hbm, out_hbm, *refs, b_per_w):
    idx_v = refs[0]
    bufs = refs[1:1 + NBUF]
    gsems = refs[1 + NBUF:1 + 2 * NBUF]
    ssems = refs[1 + 2 * NBUF:1 + 3 * NBUF]

    wid = lax.axis_index("s") * NUM_CORES + lax.axis_index("c")
    base = wid * b_per_w
    pltpu.sync_copy(idx_hbm.at[pl.ds(base, b_per_w)], idx_v)

    n = b_per_w // CHUNK
    n_grp = n // NBUF

    def gather(b, j):
        return pltpu.make_async_copy(
            table_hbm.at[idx_v.at[pl.ds(j * CHUNK, CHUNK)]], bufs[b], gsems[b])

    def store(b, j):
        return pltpu.make_async_copy(
            bufs[b], out_hbm.at[pl.ds(base + j * CHUNK, CHUNK)], ssems[b])

    def step(j, b, may_wait_store, may_fire_ahead):
        if may_wait_store:
            store((b - LOOK) % NBUF, j - LOOK).wait()
        if may_fire_ahead:
            gather((b + LOOK) % NBUF, j + LOOK).start()
        gather(b, j).wait()
        store(b, j).start()

    for j in range(LOOK):
        gather(j % NBUF, j).start()

    for b in range(NBUF):
        step(b, b, b - LOOK >= 0, b + LOOK < n)

    def grp(g, c):
        for b in range(NBUF):
            step(g * NBUF + b, b, True, True)
        return c

    lax.fori_loop(1, n_grp - 1, grp, 0)

    for b in range(NBUF):
        j = (n_grp - 1) * NBUF + b
        step(j, b, True, j + LOOK < n)

    for j in range(n - LOOK, n):
        store(j % NBUF, j).wait()


def kernel(x, weight):
    lead_shape = x.shape
    d = weight.shape[1]
    flat = x.reshape(-1).astype(jnp.int32)
    b = flat.shape[0]
    assert b % NUM_WORKERS == 0
    b_per_w = b // NUM_WORKERS
    assert b_per_w % (CHUNK * NBUF) == 0

    mesh = plsc.VectorSubcoreMesh(core_axis_name="c", subcore_axis_name="s")
    scratch = (
        [pltpu.VMEM((b_per_w,), jnp.int32)]
        + [pltpu.VMEM((CHUNK, d), jnp.float32) for _ in range(NBUF)]
        + [pltpu.SemaphoreType.DMA for _ in range(2 * NBUF)]
    )
    k = pl.kernel(
        functools.partial(_body, b_per_w=b_per_w),
        out_type=jax.ShapeDtypeStruct((b, d), jnp.float32),
        mesh=mesh,
        scratch_types=scratch,
    )
    out = k(flat, weight)
    return out.reshape(lead_shape + (d,))


# speedup vs baseline: 10.4781x; 3.1428x over previous
import functools

import jax
import jax.numpy as jnp
from jax import lax
from jax.experimental import pallas as pl
from jax.experimental.pallas import tpu as pltpu
from jax.experimental.pallas import tpu_sc as plsc

NUM_CORES = 2
NUM_SUBCORES = 16
NUM_WORKERS = NUM_CORES * NUM_SUBCORES

CHUNK = 200
NBUF = 4
LOOK = 2


def _body(idx_hbm, table_hbm, out_hbm, *refs, b_per_w):
    idx_v = refs[0]
    bufs = refs[1:1 + NBUF]
    gsems = refs[1 + NBUF:1 + 2 * NBUF]
    ssems = refs[1 + 2 * NBUF:1 + 3 * NBUF]

    wid = lax.axis_index("s") * NUM_CORES + lax.axis_index("c")
    base = wid * b_per_w
    pltpu.sync_copy(idx_hbm.at[pl.ds(base, b_per_w)], idx_v)

    n = b_per_w // CHUNK
    n_grp = n // NBUF

    def gather(b, j):
        return pltpu.make_async_copy(
            table_hbm.at[idx_v.at[pl.ds(j * CHUNK, CHUNK)]], bufs[b], gsems[b])

    def store(b, j):
        return pltpu.make_async_copy(
            bufs[b], out_hbm.at[pl.ds(base + j * CHUNK, CHUNK)], ssems[b])

    def step(j, b, may_wait_store, may_fire_ahead):
        if may_wait_store:
            store((b - LOOK) % NBUF, j - LOOK).wait()
        if may_fire_ahead:
            gather((b + LOOK) % NBUF, j + LOOK).start()
        gather(b, j).wait()
        store(b, j).start()

    for j in range(LOOK):
        gather(j % NBUF, j).start()

    for b in range(NBUF):
        step(b, b, b - LOOK >= 0, b + LOOK < n)

    def grp(g, c):
        for b in range(NBUF):
            step(g * NBUF + b, b, True, True)
        return c

    lax.fori_loop(1, n_grp - 1, grp, 0)

    for b in range(NBUF):
        j = (n_grp - 1) * NBUF + b
        step(j, b, True, j + LOOK < n)

    for j in range(n - LOOK, n):
        store(j % NBUF, j).wait()


def kernel(x, weight):
    n0, n1 = x.shape
    d = weight.shape[1]
    flat = x.T.reshape(-1).astype(jnp.int32)
    b = flat.shape[0]
    assert b % NUM_WORKERS == 0
    b_per_w = b // NUM_WORKERS
    assert b_per_w % (CHUNK * NBUF) == 0

    mesh = plsc.VectorSubcoreMesh(core_axis_name="c", subcore_axis_name="s")
    scratch = (
        [pltpu.VMEM((b_per_w,), jnp.int32)]
        + [pltpu.VMEM((CHUNK, d), jnp.float32) for _ in range(NBUF)]
        + [pltpu.SemaphoreType.DMA for _ in range(2 * NBUF)]
    )
    k = pl.kernel(
        functools.partial(_body, b_per_w=b_per_w),
        out_type=jax.ShapeDtypeStruct((b, d), jnp.float32),
        mesh=mesh,
        scratch_types=scratch,
    )
    out = k(flat, weight)
    return out.reshape(n1, n0, d).transpose(1, 0, 2)


# speedup vs baseline: 10.5100x; 1.0030x over previous
import functools

import jax
import jax.numpy as jnp
from jax import lax
from jax.experimental import pallas as pl
from jax.experimental.pallas import tpu as pltpu
from jax.experimental.pallas import tpu_sc as plsc

NUM_CORES = 2
NUM_SUBCORES = 16
NUM_WORKERS = NUM_CORES * NUM_SUBCORES

CHUNK = 80
NBUF = 8
LOOK = 4


def _body(idx_hbm, table_hbm, out_hbm, *refs, b_per_w):
    idx_v = refs[0]
    bufs = refs[1:1 + NBUF]
    gsems = refs[1 + NBUF:1 + 2 * NBUF]
    ssems = refs[1 + 2 * NBUF:1 + 3 * NBUF]

    wid = lax.axis_index("s") * NUM_CORES + lax.axis_index("c")
    base = wid * b_per_w
    pltpu.sync_copy(idx_hbm.at[pl.ds(base, b_per_w)], idx_v)

    n = b_per_w // CHUNK
    n_grp = n // NBUF

    def gather(b, j):
        return pltpu.make_async_copy(
            table_hbm.at[idx_v.at[pl.ds(j * CHUNK, CHUNK)]], bufs[b], gsems[b])

    def store(b, j):
        return pltpu.make_async_copy(
            bufs[b], out_hbm.at[pl.ds(base + j * CHUNK, CHUNK)], ssems[b])

    def step(j, b, may_wait_store, may_fire_ahead):
        if may_wait_store:
            store((b - LOOK) % NBUF, j - LOOK).wait()
        if may_fire_ahead:
            gather((b + LOOK) % NBUF, j + LOOK).start()
        gather(b, j).wait()
        store(b, j).start()

    for j in range(LOOK):
        gather(j % NBUF, j).start()

    for b in range(NBUF):
        step(b, b, b - LOOK >= 0, b + LOOK < n)

    def grp(g, c):
        for b in range(NBUF):
            step(g * NBUF + b, b, True, True)
        return c

    lax.fori_loop(1, n_grp - 1, grp, 0)

    for b in range(NBUF):
        j = (n_grp - 1) * NBUF + b
        step(j, b, True, j + LOOK < n)

    for j in range(n - LOOK, n):
        store(j % NBUF, j).wait()


def kernel(x, weight):
    n0, n1 = x.shape
    d = weight.shape[1]
    flat = x.T.reshape(-1).astype(jnp.int32)
    b = flat.shape[0]
    assert b % NUM_WORKERS == 0
    b_per_w = b // NUM_WORKERS
    assert b_per_w % (CHUNK * NBUF) == 0

    mesh = plsc.VectorSubcoreMesh(core_axis_name="c", subcore_axis_name="s")
    scratch = (
        [pltpu.VMEM((b_per_w,), jnp.int32)]
        + [pltpu.VMEM((CHUNK, d), jnp.float32) for _ in range(NBUF)]
        + [pltpu.SemaphoreType.DMA for _ in range(2 * NBUF)]
    )
    k = pl.kernel(
        functools.partial(_body, b_per_w=b_per_w),
        out_type=jax.ShapeDtypeStruct((b, d), jnp.float32),
        mesh=mesh,
        scratch_types=scratch,
    )
    out = k(flat, weight)
    return out.reshape(n1, n0, d).transpose(1, 0, 2)
